# Initial kernel scaffold; baseline (speedup 1.0000x reference)
import jax
import jax.numpy as jnp
from jax.experimental import pallas as pl


def kernel(pos, params, batch):
    raise NotImplementedError("write your pallas kernel here")


# speedup vs baseline: 5.3799x; 5.3799x over previous
import functools

import jax
import jax.numpy as jnp
from jax import lax
from jax.experimental import pallas as pl
from jax.experimental.pallas import tpu as pltpu
from jax.experimental.pallas import tpu_sc as plsc

N = 8192
B = 8
K = 20
DP = 128

_bf = jnp.bfloat16


def _mm(a, b):
    return jnp.dot(a.astype(_bf), b.astype(_bf),
                   preferred_element_type=jnp.float32)


def _p_body(x_ref, w1a_ref, b1_ref, p_ref, sq_ref):
    x = x_ref[...]
    p_ref[...] = _mm(x, w1a_ref[...]) + b1_ref[...]
    sq_ref[...] = jnp.sum(x * x, axis=1, keepdims=True)


def _p_and_sq(x, w1a, b1):
    n = x.shape[0]
    h = w1a.shape[1]
    rb = 1024
    return pl.pallas_call(
        _p_body,
        grid=(n // rb,),
        in_specs=[
            pl.BlockSpec((rb, DP), lambda i: (i, 0)),
            pl.BlockSpec((DP, h), lambda i: (0, 0)),
            pl.BlockSpec((1, h), lambda i: (0, 0)),
        ],
        out_specs=[
            pl.BlockSpec((rb, h), lambda i: (i, 0)),
            pl.BlockSpec((rb, 1), lambda i: (i, 0)),
        ],
        out_shape=[
            jax.ShapeDtypeStruct((n, h), jnp.float32),
            jax.ShapeDtypeStruct((n, 1), jnp.float32),
        ],
    )(x, w1a, b1)


def _knn_body(xr_ref, xt_ref, sqr_ref, sqc_ref, br_ref, bc_ref, idx_ref):
    dot = _mm(xr_ref[...], xt_ref[...])
    dist = sqr_ref[...] + sqc_ref[...] - 2.0 * dot
    dist = jnp.where(br_ref[...] != bc_ref[...], jnp.inf, dist)
    cols = lax.broadcasted_iota(jnp.int32, dist.shape, 1)
    picks = []
    for _ in range(K):
        m = jnp.min(dist, axis=1, keepdims=True)
        cand = jnp.where(dist == m, cols, N)
        j = jnp.min(cand, axis=1, keepdims=True)
        picks.append(j)
        dist = jnp.where(cols == j, jnp.inf, dist)
    idx_ref[...] = jnp.concatenate(picks, axis=1)


def _knn(x, xt, sq_row, sq_col, b_row, b_col):
    n = x.shape[0]
    rb = 256
    return pl.pallas_call(
        _knn_body,
        grid=(n // rb,),
        in_specs=[
            pl.BlockSpec((rb, DP), lambda i: (i, 0)),
            pl.BlockSpec((DP, n), lambda i: (0, 0)),
            pl.BlockSpec((rb, 1), lambda i: (i, 0)),
            pl.BlockSpec((1, n), lambda i: (0, 0)),
            pl.BlockSpec((rb, 1), lambda i: (i, 0)),
            pl.BlockSpec((1, n), lambda i: (0, 0)),
        ],
        out_specs=pl.BlockSpec((rb, K), lambda i: (i, 0)),
        out_shape=jax.ShapeDtypeStruct((n, K), jnp.int32),
    )(x, xt, sq_row, sq_col, b_row, b_col)


def _sc_gather(table, idx_flat):
    n, h = table.shape
    m = idx_flat.shape[0]
    nw = 32
    per_w = m // nw
    rows_per_chunk = min(per_w, max(8, (128 * 1024) // (h * 4)))
    n_chunks = per_w // rows_per_chunk
    mesh = plsc.VectorSubcoreMesh(core_axis_name="c", subcore_axis_name="s")

    @functools.partial(
        pl.kernel,
        mesh=mesh,
        out_type=jax.ShapeDtypeStruct((m, h), jnp.float32),
        scratch_types=[
            pltpu.VMEM((rows_per_chunk,), jnp.int32),
            pltpu.VMEM((rows_per_chunk, h), jnp.float32),
            pltpu.SemaphoreType.DMA,
        ],
    )
    def k(tab_hbm, idx_hbm, out_hbm, idx_v, rows_v, sem):
        wid = lax.axis_index("s") * 2 + lax.axis_index("c")
        base = wid * per_w

        def body(c, carry):
            off = base + c * rows_per_chunk
            pltpu.sync_copy(idx_hbm.at[pl.ds(off, rows_per_chunk)], idx_v)
            pltpu.async_copy(tab_hbm.at[idx_v], rows_v, sem).wait()
            pltpu.sync_copy(rows_v, out_hbm.at[pl.ds(off, rows_per_chunk)])
            return carry

        lax.fori_loop(0, n_chunks, body, 0)

    return k(table, idx_flat)


def _mlp_body(x_ref, xg_ref, p_ref, w1b_ref, w2_ref, b2_ref, o_ref):
    rb = x_ref.shape[0]
    h = p_ref.shape[1]
    o_dim = w2_ref.shape[1]
    delta = xg_ref[...].reshape(rb, K, DP) - x_ref[...][:, None, :]
    hid = _mm(delta.reshape(rb * K, DP), w1b_ref[...]).reshape(rb, K, h)
    hid = jnp.maximum(hid + p_ref[...][:, None, :], 0.0)
    hh = _mm(hid.reshape(rb * K, h), w2_ref[...])
    o_ref[...] = jnp.max(hh.reshape(rb, K, o_dim), axis=1) + b2_ref[...]


def _mlp(x, xg, p, w1b, w2, b2):
    n, h = p.shape
    o_dim = w2.shape[1]
    rb = 128
    return pl.pallas_call(
        _mlp_body,
        grid=(n // rb,),
        in_specs=[
            pl.BlockSpec((rb, DP), lambda i: (i, 0)),
            pl.BlockSpec((rb * K, DP), lambda i: (i, 0)),
            pl.BlockSpec((rb, h), lambda i: (i, 0)),
            pl.BlockSpec((DP, h), lambda i: (0, 0)),
            pl.BlockSpec((h, o_dim), lambda i: (0, 0)),
            pl.BlockSpec((1, o_dim), lambda i: (0, 0)),
        ],
        out_specs=pl.BlockSpec((rb, o_dim), lambda i: (i, 0)),
        out_shape=jax.ShapeDtypeStruct((n, o_dim), jnp.float32),
    )(x, xg, p, w1b, w2, b2)


def _head_body(x1_ref, x2_ref, x3_ref, x4_ref, bt_ref, w_ref, bv_ref,
               g_ref, be_ref, w2_ref, b2_ref, out_ref):
    xcat = jnp.concatenate(
        [x1_ref[...], x2_ref[...], x3_ref[...], x4_ref[...]], axis=1)
    bt = bt_ref[...]
    pooled = []
    for seg in range(B):
        pooled.append(
            jnp.max(jnp.where(bt == seg, xcat, -jnp.inf), axis=0,
                    keepdims=True))
    pooled = jnp.concatenate(pooled, axis=0)
    hh = _mm(pooled, w_ref[...]) + bv_ref[...]
    mu = jnp.mean(hh, axis=0, keepdims=True)
    var = jnp.mean((hh - mu) * (hh - mu), axis=0, keepdims=True)
    hn = g_ref[...] * (hh - mu) / jnp.sqrt(var + 1e-5) + be_ref[...]
    hn = jnp.maximum(hn, 0.0)
    logits = _mm(hn, w2_ref[...]) + b2_ref[...]
    mx = jnp.max(logits, axis=1, keepdims=True)
    sh = logits - mx
    out_ref[...] = sh - jnp.log(jnp.sum(jnp.exp(sh), axis=1, keepdims=True))


def _head(x1, x2, x3, x4, bt, w, bv, g, be, w2, b2):
    nc = w2.shape[1]
    emb = w.shape[1]
    return pl.pallas_call(
        _head_body,
        in_specs=[pl.BlockSpec(x.shape, lambda: (0, 0))
                  for x in (x1, x2, x3, x4)]
        + [
            pl.BlockSpec((N, 1), lambda: (0, 0)),
            pl.BlockSpec((512, emb), lambda: (0, 0)),
            pl.BlockSpec((1, emb), lambda: (0, 0)),
            pl.BlockSpec((1, emb), lambda: (0, 0)),
            pl.BlockSpec((1, emb), lambda: (0, 0)),
            pl.BlockSpec((emb, nc), lambda: (0, 0)),
            pl.BlockSpec((1, nc), lambda: (0, 0)),
        ],
        out_specs=pl.BlockSpec((B, nc), lambda: (0, 0)),
        out_shape=jax.ShapeDtypeStruct((B, nc), jnp.float32),
    )(x1, x2, x3, x4, bt, w, bv, g, be, w2, b2)


def _edge_conv(x, b_row, b_col, p_conv):
    w1 = p_conv["W1"]
    d = x.shape[1]
    w1a, w1b = w1[:d], w1[d:]
    if d != DP:
        x = jnp.pad(x, ((0, 0), (0, DP - d)))
        w1a = jnp.pad(w1a, ((0, DP - d), (0, 0)))
        w1b = jnp.pad(w1b, ((0, DP - d), (0, 0)))
    p, sq = _p_and_sq(x, w1a, p_conv["b1"][None, :])
    idx = _knn(x, x.T, sq, sq.reshape(1, N), b_row, b_col)
    xg = _sc_gather(x, idx.reshape(N * K))
    return _mlp(x, xg, p, w1b, p_conv["W2"], p_conv["b2"][None, :])


def kernel(pos, params, batch):
    bt = batch.astype(jnp.int32)
    b_row = bt.reshape(N, 1)
    b_col = bt.reshape(1, N)
    x1 = _edge_conv(pos, b_row, b_col, params["conv1"])
    x2 = _edge_conv(x1, b_row, b_col, params["conv2"])
    x3 = _edge_conv(x2, b_row, b_col, params["conv3"])
    x4 = _edge_conv(x3, b_row, b_col, params["conv4"])
    return _head(
        x1, x2, x3, x4, b_row,
        params["lin1"]["W"], params["lin1"]["b"][None, :],
        params["bn1"]["gamma"][None, :], params["bn1"]["beta"][None, :],
        params["lin2"]["W"], params["lin2"]["b"][None, :],
    )


# speedup vs baseline: 11.6287x; 2.1615x over previous
import functools

import jax
import jax.numpy as jnp
from jax import lax
from jax.experimental import pallas as pl
from jax.experimental.pallas import tpu as pltpu
from jax.experimental.pallas import tpu_sc as plsc

N = 8192
B = 8
K = 20
DP = 128

_bf = jnp.bfloat16


def _mm(a, b):
    return jnp.dot(a.astype(_bf), b.astype(_bf),
                   preferred_element_type=jnp.float32)


def _p_body(x_ref, w1a_ref, b1_ref, p_ref, sq_ref):
    x = x_ref[...]
    p_ref[...] = _mm(x, w1a_ref[...]) + b1_ref[...]
    sq_ref[...] = jnp.sum(x * x, axis=1, keepdims=True)


def _p_and_sq(x, w1a, b1):
    n = x.shape[0]
    h = w1a.shape[1]
    rb = 1024
    return pl.pallas_call(
        _p_body,
        grid=(n // rb,),
        in_specs=[
            pl.BlockSpec((rb, DP), lambda i: (i, 0)),
            pl.BlockSpec((DP, h), lambda i: (0, 0)),
            pl.BlockSpec((1, h), lambda i: (0, 0)),
        ],
        out_specs=[
            pl.BlockSpec((rb, h), lambda i: (i, 0)),
            pl.BlockSpec((rb, 1), lambda i: (i, 0)),
        ],
        out_shape=[
            jax.ShapeDtypeStruct((n, h), jnp.float32),
            jax.ShapeDtypeStruct((n, 1), jnp.float32),
        ],
    )(x, w1a, b1)


_CS = 512


def _topk_merge(tv, ti, cv, cc):
    picks_v, picks_i = [], []
    for _ in range(K):
        m = jnp.minimum(jnp.min(tv, axis=1, keepdims=True),
                        jnp.min(cv, axis=1, keepdims=True))
        jt = jnp.min(jnp.where(tv == m, ti, N), axis=1, keepdims=True)
        jc = jnp.min(jnp.where(cv == m, cc, N), axis=1, keepdims=True)
        j = jnp.minimum(jt, jc)
        picks_v.append(m)
        picks_i.append(j)
        tv = jnp.where(ti == j, jnp.inf, tv)
        cv = jnp.where(cc == j, jnp.inf, cv)
    return (jnp.concatenate(picks_v, axis=1),
            jnp.concatenate(picks_i, axis=1))


def _knn_body(cs_ref, ce_ref, xr_ref, xt_ref, sqr_ref, sqc_ref, br_ref,
              bc_ref, idx_ref):
    i = pl.program_id(0)
    rb = xr_ref.shape[0]
    xr = xr_ref[...]
    sqr = sqr_ref[...]
    br = br_ref[...]
    pcols = lax.broadcasted_iota(jnp.int32, (rb, 2 * K), 1)
    p_out = bc_ref[:, : 2 * K] != br
    pv = jnp.full((rb, 2 * K), jnp.inf, jnp.float32)
    pc = jnp.where(p_out, pcols, N)
    tv0 = jnp.full((rb, K), jnp.inf, jnp.float32)
    ti0 = jnp.full((rb, K), N, jnp.int32)
    tv, ti = _topk_merge(tv0, ti0, pv, pc)

    def body(c, carry):
        tv, ti = carry
        col0 = c * _CS
        xc = xt_ref[:, pl.ds(col0, _CS)]
        dot = jnp.dot(xr, xc, preferred_element_type=jnp.float32)
        dval = sqr + sqc_ref[:, pl.ds(col0, _CS)] - 2.0 * dot
        dval = jnp.where(br != bc_ref[:, pl.ds(col0, _CS)], jnp.inf, dval)
        dcol = lax.broadcasted_iota(jnp.int32, (rb, _CS), 1) + col0
        return _topk_merge(tv, ti, dval, dcol)

    tv, ti = lax.fori_loop(cs_ref[i], ce_ref[i], body, (tv, ti))
    idx_ref[...] = ti


def _knn(x_bf, xt_bf, cs, ce, sq_row, sq_col, b_row, b_col):
    n = x_bf.shape[0]
    rb = 256
    return pl.pallas_call(
        _knn_body,
        grid=(n // rb,),
        in_specs=[
            pl.BlockSpec(memory_space=pltpu.SMEM),
            pl.BlockSpec(memory_space=pltpu.SMEM),
            pl.BlockSpec((rb, DP), lambda i: (i, 0)),
            pl.BlockSpec((DP, n), lambda i: (0, 0)),
            pl.BlockSpec((rb, 1), lambda i: (i, 0)),
            pl.BlockSpec((1, n), lambda i: (0, 0)),
            pl.BlockSpec((rb, 1), lambda i: (i, 0)),
            pl.BlockSpec((1, n), lambda i: (0, 0)),
        ],
        out_specs=pl.BlockSpec((rb, K), lambda i: (i, 0)),
        out_shape=jax.ShapeDtypeStruct((n, K), jnp.int32),
    )(cs, ce, x_bf, xt_bf, sq_row, sq_col, b_row, b_col)


def _sc_gather(table, idx_flat):
    n, h = table.shape
    m = idx_flat.shape[0]
    nw = 32
    per_w = m // nw
    rows_per_chunk = min(per_w, max(8, (128 * 1024) // (h * 4)))
    n_chunks = per_w // rows_per_chunk
    mesh = plsc.VectorSubcoreMesh(core_axis_name="c", subcore_axis_name="s")

    @functools.partial(
        pl.kernel,
        mesh=mesh,
        out_type=jax.ShapeDtypeStruct((m, h), jnp.float32),
        scratch_types=[
            pltpu.VMEM((rows_per_chunk,), jnp.int32),
            pltpu.VMEM((rows_per_chunk, h), jnp.float32),
            pltpu.SemaphoreType.DMA,
        ],
    )
    def k(tab_hbm, idx_hbm, out_hbm, idx_v, rows_v, sem):
        wid = lax.axis_index("s") * 2 + lax.axis_index("c")
        base = wid * per_w

        def body(c, carry):
            off = base + c * rows_per_chunk
            pltpu.sync_copy(idx_hbm.at[pl.ds(off, rows_per_chunk)], idx_v)
            pltpu.async_copy(tab_hbm.at[idx_v], rows_v, sem).wait()
            pltpu.sync_copy(rows_v, out_hbm.at[pl.ds(off, rows_per_chunk)])
            return carry

        lax.fori_loop(0, n_chunks, body, 0)

    return k(table, idx_flat)


def _mlp_body(x_ref, xg_ref, p_ref, w1b_ref, w2_ref, b2_ref, o_ref):
    rb = x_ref.shape[0]
    h = p_ref.shape[1]
    o_dim = w2_ref.shape[1]
    delta = xg_ref[...].reshape(rb, K, DP) - x_ref[...][:, None, :]
    hid = _mm(delta.reshape(rb * K, DP), w1b_ref[...]).reshape(rb, K, h)
    hid = jnp.maximum(hid + p_ref[...][:, None, :], 0.0)
    hh = _mm(hid.reshape(rb * K, h), w2_ref[...])
    o_ref[...] = jnp.max(hh.reshape(rb, K, o_dim), axis=1) + b2_ref[...]


def _mlp(x, xg, p, w1b, w2, b2):
    n, h = p.shape
    o_dim = w2.shape[1]
    rb = 128
    return pl.pallas_call(
        _mlp_body,
        grid=(n // rb,),
        in_specs=[
            pl.BlockSpec((rb, DP), lambda i: (i, 0)),
            pl.BlockSpec((rb * K, DP), lambda i: (i, 0)),
            pl.BlockSpec((rb, h), lambda i: (i, 0)),
            pl.BlockSpec((DP, h), lambda i: (0, 0)),
            pl.BlockSpec((h, o_dim), lambda i: (0, 0)),
            pl.BlockSpec((1, o_dim), lambda i: (0, 0)),
        ],
        out_specs=pl.BlockSpec((rb, o_dim), lambda i: (i, 0)),
        out_shape=jax.ShapeDtypeStruct((n, o_dim), jnp.float32),
    )(x, xg, p, w1b, w2, b2)


def _head_body(x1_ref, x2_ref, x3_ref, x4_ref, bt_ref, w_ref, bv_ref,
               g_ref, be_ref, w2_ref, b2_ref, out_ref):
    xcat = jnp.concatenate(
        [x1_ref[...], x2_ref[...], x3_ref[...], x4_ref[...]], axis=1)
    bt = bt_ref[...]
    pooled = []
    for seg in range(B):
        pooled.append(
            jnp.max(jnp.where(bt == seg, xcat, -jnp.inf), axis=0,
                    keepdims=True))
    pooled = jnp.concatenate(pooled, axis=0)
    hh = _mm(pooled, w_ref[...]) + bv_ref[...]
    mu = jnp.mean(hh, axis=0, keepdims=True)
    var = jnp.mean((hh - mu) * (hh - mu), axis=0, keepdims=True)
    hn = g_ref[...] * (hh - mu) / jnp.sqrt(var + 1e-5) + be_ref[...]
    hn = jnp.maximum(hn, 0.0)
    logits = _mm(hn, w2_ref[...]) + b2_ref[...]
    mx = jnp.max(logits, axis=1, keepdims=True)
    sh = logits - mx
    out_ref[...] = sh - jnp.log(jnp.sum(jnp.exp(sh), axis=1, keepdims=True))


def _head(x1, x2, x3, x4, bt, w, bv, g, be, w2, b2):
    nc = w2.shape[1]
    emb = w.shape[1]
    return pl.pallas_call(
        _head_body,
        in_specs=[pl.BlockSpec(x.shape, lambda: (0, 0))
                  for x in (x1, x2, x3, x4)]
        + [
            pl.BlockSpec((N, 1), lambda: (0, 0)),
            pl.BlockSpec((512, emb), lambda: (0, 0)),
            pl.BlockSpec((1, emb), lambda: (0, 0)),
            pl.BlockSpec((1, emb), lambda: (0, 0)),
            pl.BlockSpec((1, emb), lambda: (0, 0)),
            pl.BlockSpec((emb, nc), lambda: (0, 0)),
            pl.BlockSpec((1, nc), lambda: (0, 0)),
        ],
        out_specs=pl.BlockSpec((B, nc), lambda: (0, 0)),
        out_shape=jax.ShapeDtypeStruct((B, nc), jnp.float32),
    )(x1, x2, x3, x4, bt, w, bv, g, be, w2, b2)


def _edge_conv(x, seg, p_conv):
    b_row, b_col, cs, ce = seg
    w1 = p_conv["W1"]
    d = x.shape[1]
    w1a, w1b = w1[:d], w1[d:]
    if d != DP:
        x = jnp.pad(x, ((0, 0), (0, DP - d)))
        w1a = jnp.pad(w1a, ((0, DP - d), (0, 0)))
        w1b = jnp.pad(w1b, ((0, DP - d), (0, 0)))
    p, sq = _p_and_sq(x, w1a, p_conv["b1"][None, :])
    x_bf = x.astype(_bf)
    idx = _knn(x_bf, x_bf.T, cs, ce, sq, sq.reshape(1, N), b_row, b_col)
    xg = _sc_gather(x, idx.reshape(N * K))
    return _mlp(x, xg, p, w1b, p_conv["W2"], p_conv["b2"][None, :])


def kernel(pos, params, batch):
    bt = batch.astype(jnp.int32)
    b_row = bt.reshape(N, 1)
    b_col = bt.reshape(1, N)
    segs = jnp.arange(B, dtype=jnp.int32)
    starts = jnp.searchsorted(bt, segs, side="left")
    ends = jnp.searchsorted(bt, segs, side="right")
    cs = (starts[bt[::256]] // _CS).astype(jnp.int32)
    ce = ((ends[bt[255::256]] + _CS - 1) // _CS).astype(jnp.int32)
    seg = (b_row, b_col, cs, ce)
    x1 = _edge_conv(pos, seg, params["conv1"])
    x2 = _edge_conv(x1, seg, params["conv2"])
    x3 = _edge_conv(x2, seg, params["conv3"])
    x4 = _edge_conv(x3, seg, params["conv4"])
    return _head(
        x1, x2, x3, x4, b_row,
        params["lin1"]["W"], params["lin1"]["b"][None, :],
        params["bn1"]["gamma"][None, :], params["bn1"]["beta"][None, :],
        params["lin2"]["W"], params["lin2"]["b"][None, :],
    )


# speedup vs baseline: 17.0094x; 1.4627x over previous
import functools

import jax
import jax.numpy as jnp
from jax import lax
from jax.experimental import pallas as pl
from jax.experimental.pallas import tpu as pltpu
from jax.experimental.pallas import tpu_sc as plsc

N = 8192
B = 8
K = 20
DP = 128

_bf = jnp.bfloat16


def _mm(a, b):
    return jnp.dot(a.astype(_bf), b.astype(_bf),
                   preferred_element_type=jnp.float32)


def _p_body(x_ref, w1a_ref, b1_ref, p_ref, sq_ref):
    x = x_ref[...]
    p_ref[...] = _mm(x, w1a_ref[...]) + b1_ref[...]
    sq_ref[...] = jnp.sum(x * x, axis=1, keepdims=True)


def _p_and_sq(x, w1a, b1):
    n = x.shape[0]
    h = w1a.shape[1]
    rb = 1024
    return pl.pallas_call(
        _p_body,
        grid=(n // rb,),
        in_specs=[
            pl.BlockSpec((rb, DP), lambda i: (i, 0)),
            pl.BlockSpec((DP, h), lambda i: (0, 0)),
            pl.BlockSpec((1, h), lambda i: (0, 0)),
        ],
        out_specs=[
            pl.BlockSpec((rb, h), lambda i: (i, 0)),
            pl.BlockSpec((rb, 1), lambda i: (i, 0)),
        ],
        out_shape=[
            jax.ShapeDtypeStruct((n, h), jnp.float32),
            jax.ShapeDtypeStruct((n, 1), jnp.float32),
        ],
    )(x, w1a, b1)


_CS = 512


_FN = float(N)


def _topk_merge(tv, ti, cv, cc):
    picks_v, picks_i = [], []
    for _ in range(K):
        m = jnp.minimum(jnp.min(tv, axis=1, keepdims=True),
                        jnp.min(cv, axis=1, keepdims=True))
        jt = jnp.min(jnp.where(tv == m, ti, _FN), axis=1, keepdims=True)
        jc = jnp.min(jnp.where(cv == m, cc, _FN), axis=1, keepdims=True)
        j = jnp.minimum(jt, jc)
        picks_v.append(m)
        picks_i.append(j)
        tv = jnp.where(ti == j, jnp.inf, tv)
        cv = jnp.where(cc == j, jnp.inf, cv)
    return (jnp.concatenate(picks_v, axis=1),
            jnp.concatenate(picks_i, axis=1))


def _knn_body(cs_ref, ce_ref, xr_ref, xt_ref, sqr_ref, sqc_ref, br_ref,
              bc_ref, idx_ref):
    i = pl.program_id(0)
    rb = xr_ref.shape[0]
    xr = xr_ref[...]
    sqr = sqr_ref[...]
    br = br_ref[...]
    pcols = lax.broadcasted_iota(jnp.int32, (rb, 2 * K), 1).astype(jnp.float32)
    p_out = bc_ref[:, : 2 * K] != br
    pv = jnp.full((rb, 2 * K), jnp.inf, jnp.float32)
    pc = jnp.where(p_out, pcols, _FN)
    tv0 = jnp.full((rb, K), jnp.inf, jnp.float32)
    ti0 = jnp.full((rb, K), _FN, jnp.float32)
    tv, ti = _topk_merge(tv0, ti0, pv, pc)

    def body(c, carry):
        tv, ti = carry
        col0 = c * _CS
        xc = xt_ref[:, pl.ds(col0, _CS)]
        dot = jnp.dot(xr, xc, preferred_element_type=jnp.float32)
        dval = sqr + sqc_ref[:, pl.ds(col0, _CS)] - 2.0 * dot
        dval = jnp.where(br != bc_ref[:, pl.ds(col0, _CS)], jnp.inf, dval)
        dcol = (lax.broadcasted_iota(jnp.int32, (rb, _CS), 1).astype(jnp.float32)
                + col0.astype(jnp.float32))
        return _topk_merge(tv, ti, dval, dcol)

    tv, ti = lax.fori_loop(cs_ref[i], ce_ref[i], body, (tv, ti))
    idx_ref[...] = ti.astype(jnp.int32)


def _knn(x_bf, xt_bf, cs, ce, sq_row, sq_col, b_row, b_col):
    n = x_bf.shape[0]
    rb = 256
    return pl.pallas_call(
        _knn_body,
        grid=(n // rb,),
        in_specs=[
            pl.BlockSpec(memory_space=pltpu.SMEM),
            pl.BlockSpec(memory_space=pltpu.SMEM),
            pl.BlockSpec((rb, DP), lambda i: (i, 0)),
            pl.BlockSpec((DP, n), lambda i: (0, 0)),
            pl.BlockSpec((rb, 1), lambda i: (i, 0)),
            pl.BlockSpec((1, n), lambda i: (0, 0)),
            pl.BlockSpec((rb, 1), lambda i: (i, 0)),
            pl.BlockSpec((1, n), lambda i: (0, 0)),
        ],
        out_specs=pl.BlockSpec((rb, K), lambda i: (i, 0)),
        out_shape=jax.ShapeDtypeStruct((n, K), jnp.int32),
    )(cs, ce, x_bf, xt_bf, sq_row, sq_col, b_row, b_col)


def _sc_gather(table, idx_flat):
    n, h = table.shape
    m = idx_flat.shape[0]
    nw = 32
    per_w = m // nw
    rows_per_chunk = min(per_w, max(8, (128 * 1024) // (h * 4)))
    n_chunks = per_w // rows_per_chunk
    mesh = plsc.VectorSubcoreMesh(core_axis_name="c", subcore_axis_name="s")

    @functools.partial(
        pl.kernel,
        mesh=mesh,
        out_type=jax.ShapeDtypeStruct((m, h), jnp.float32),
        scratch_types=[
            pltpu.VMEM((rows_per_chunk,), jnp.int32),
            pltpu.VMEM((rows_per_chunk, h), jnp.float32),
            pltpu.SemaphoreType.DMA,
        ],
    )
    def k(tab_hbm, idx_hbm, out_hbm, idx_v, rows_v, sem):
        wid = lax.axis_index("s") * 2 + lax.axis_index("c")
        base = wid * per_w

        def body(c, carry):
            off = base + c * rows_per_chunk
            pltpu.sync_copy(idx_hbm.at[pl.ds(off, rows_per_chunk)], idx_v)
            pltpu.async_copy(tab_hbm.at[idx_v], rows_v, sem).wait()
            pltpu.sync_copy(rows_v, out_hbm.at[pl.ds(off, rows_per_chunk)])
            return carry

        lax.fori_loop(0, n_chunks, body, 0)

    return k(table, idx_flat)


def _mlp_body(x_ref, xg_ref, p_ref, w1b_ref, w2_ref, b2_ref, o_ref):
    rb = x_ref.shape[0]
    h = p_ref.shape[1]
    o_dim = w2_ref.shape[1]
    delta = xg_ref[...].reshape(rb, K, DP) - x_ref[...][:, None, :]
    hid = _mm(delta.reshape(rb * K, DP), w1b_ref[...]).reshape(rb, K, h)
    hid = jnp.maximum(hid + p_ref[...][:, None, :], 0.0)
    hh = _mm(hid.reshape(rb * K, h), w2_ref[...])
    o_ref[...] = jnp.max(hh.reshape(rb, K, o_dim), axis=1) + b2_ref[...]


def _mlp(x, xg, p, w1b, w2, b2):
    n, h = p.shape
    o_dim = w2.shape[1]
    rb = 128
    return pl.pallas_call(
        _mlp_body,
        grid=(n // rb,),
        in_specs=[
            pl.BlockSpec((rb, DP), lambda i: (i, 0)),
            pl.BlockSpec((rb * K, DP), lambda i: (i, 0)),
            pl.BlockSpec((rb, h), lambda i: (i, 0)),
            pl.BlockSpec((DP, h), lambda i: (0, 0)),
            pl.BlockSpec((h, o_dim), lambda i: (0, 0)),
            pl.BlockSpec((1, o_dim), lambda i: (0, 0)),
        ],
        out_specs=pl.BlockSpec((rb, o_dim), lambda i: (i, 0)),
        out_shape=jax.ShapeDtypeStruct((n, o_dim), jnp.float32),
    )(x, xg, p, w1b, w2, b2)


def _head_body(x1_ref, x2_ref, x3_ref, x4_ref, bt_ref, w_ref, bv_ref,
               g_ref, be_ref, w2_ref, b2_ref, out_ref):
    xcat = jnp.concatenate(
        [x1_ref[...], x2_ref[...], x3_ref[...], x4_ref[...]], axis=1)
    bt = bt_ref[...]
    pooled = []
    for seg in range(B):
        pooled.append(
            jnp.max(jnp.where(bt == seg, xcat, -jnp.inf), axis=0,
                    keepdims=True))
    pooled = jnp.concatenate(pooled, axis=0)
    hh = _mm(pooled, w_ref[...]) + bv_ref[...]
    mu = jnp.mean(hh, axis=0, keepdims=True)
    var = jnp.mean((hh - mu) * (hh - mu), axis=0, keepdims=True)
    hn = g_ref[...] * (hh - mu) / jnp.sqrt(var + 1e-5) + be_ref[...]
    hn = jnp.maximum(hn, 0.0)
    logits = _mm(hn, w2_ref[...]) + b2_ref[...]
    mx = jnp.max(logits, axis=1, keepdims=True)
    sh = logits - mx
    out_ref[...] = sh - jnp.log(jnp.sum(jnp.exp(sh), axis=1, keepdims=True))


def _head(x1, x2, x3, x4, bt, w, bv, g, be, w2, b2):
    nc = w2.shape[1]
    emb = w.shape[1]
    return pl.pallas_call(
        _head_body,
        in_specs=[pl.BlockSpec(x.shape, lambda: (0, 0))
                  for x in (x1, x2, x3, x4)]
        + [
            pl.BlockSpec((N, 1), lambda: (0, 0)),
            pl.BlockSpec((512, emb), lambda: (0, 0)),
            pl.BlockSpec((1, emb), lambda: (0, 0)),
            pl.BlockSpec((1, emb), lambda: (0, 0)),
            pl.BlockSpec((1, emb), lambda: (0, 0)),
            pl.BlockSpec((emb, nc), lambda: (0, 0)),
            pl.BlockSpec((1, nc), lambda: (0, 0)),
        ],
        out_specs=pl.BlockSpec((B, nc), lambda: (0, 0)),
        out_shape=jax.ShapeDtypeStruct((B, nc), jnp.float32),
    )(x1, x2, x3, x4, bt, w, bv, g, be, w2, b2)


def _edge_conv(x, seg, p_conv):
    b_row, b_col, cs, ce = seg
    w1 = p_conv["W1"]
    d = x.shape[1]
    w1a, w1b = w1[:d], w1[d:]
    if d != DP:
        x = jnp.pad(x, ((0, 0), (0, DP - d)))
        w1a = jnp.pad(w1a, ((0, DP - d), (0, 0)))
        w1b = jnp.pad(w1b, ((0, DP - d), (0, 0)))
    p, sq = _p_and_sq(x, w1a, p_conv["b1"][None, :])
    x_bf = x.astype(_bf)
    idx = _knn(x_bf, x_bf.T, cs, ce, sq, sq.reshape(1, N), b_row, b_col)
    xg = _sc_gather(x, idx.reshape(N * K))
    return _mlp(x, xg, p, w1b, p_conv["W2"], p_conv["b2"][None, :])


def kernel(pos, params, batch):
    bt = batch.astype(jnp.int32)
    b_row = bt.reshape(N, 1)
    b_col = bt.reshape(1, N)
    segs = jnp.arange(B, dtype=jnp.int32)
    starts = jnp.searchsorted(bt, segs, side="left")
    ends = jnp.searchsorted(bt, segs, side="right")
    cs = (starts[bt[::256]] // _CS).astype(jnp.int32)
    ce = ((ends[bt[255::256]] + _CS - 1) // _CS).astype(jnp.int32)
    seg = (b_row, b_col, cs, ce)
    x1 = _edge_conv(pos, seg, params["conv1"])
    x2 = _edge_conv(x1, seg, params["conv2"])
    x3 = _edge_conv(x2, seg, params["conv3"])
    x4 = _edge_conv(x3, seg, params["conv4"])
    return _head(
        x1, x2, x3, x4, b_row,
        params["lin1"]["W"], params["lin1"]["b"][None, :],
        params["bn1"]["gamma"][None, :], params["bn1"]["beta"][None, :],
        params["lin2"]["W"], params["lin2"]["b"][None, :],
    )


# speedup vs baseline: 17.1148x; 1.0062x over previous
import functools

import jax
import jax.numpy as jnp
from jax import lax
from jax.experimental import pallas as pl
from jax.experimental.pallas import tpu as pltpu
from jax.experimental.pallas import tpu_sc as plsc

N = 8192
B = 8
K = 20
DP = 128

_bf = jnp.bfloat16


def _mm(a, b):
    return jnp.dot(a.astype(_bf), b.astype(_bf),
                   preferred_element_type=jnp.float32)


def _p_body(x_ref, w1a_ref, b1_ref, p_ref, sq_ref):
    x = x_ref[...]
    p_ref[...] = _mm(x, w1a_ref[...]) + b1_ref[...]
    sq_ref[...] = jnp.sum(x * x, axis=1, keepdims=True)


def _p_and_sq(x, w1a, b1):
    n = x.shape[0]
    h = w1a.shape[1]
    rb = 1024
    return pl.pallas_call(
        _p_body,
        grid=(n // rb,),
        in_specs=[
            pl.BlockSpec((rb, DP), lambda i: (i, 0)),
            pl.BlockSpec((DP, h), lambda i: (0, 0)),
            pl.BlockSpec((1, h), lambda i: (0, 0)),
        ],
        out_specs=[
            pl.BlockSpec((rb, h), lambda i: (i, 0)),
            pl.BlockSpec((rb, 1), lambda i: (i, 0)),
        ],
        out_shape=[
            jax.ShapeDtypeStruct((n, h), jnp.float32),
            jax.ShapeDtypeStruct((n, 1), jnp.float32),
        ],
    )(x, w1a, b1)


_CS = 512


_FN = float(N)


def _topk_merge(tv, ti, cv, cc):
    picks_v, picks_i = [], []
    for _ in range(K):
        m = jnp.minimum(jnp.min(tv, axis=1, keepdims=True),
                        jnp.min(cv, axis=1, keepdims=True))
        jt = jnp.min(jnp.where(tv == m, ti, _FN), axis=1, keepdims=True)
        jc = jnp.min(jnp.where(cv == m, cc, _FN), axis=1, keepdims=True)
        j = jnp.minimum(jt, jc)
        picks_v.append(m)
        picks_i.append(j)
        tv = jnp.where(ti == j, jnp.inf, tv)
        cv = jnp.where(cc == j, jnp.inf, cv)
    return (jnp.concatenate(picks_v, axis=1),
            jnp.concatenate(picks_i, axis=1))


def _knn_body(cs_ref, ce_ref, xr_ref, xt_ref, sqr_ref, sqc_ref, br_ref,
              bc_ref, idx_ref):
    i = pl.program_id(0)
    rb = xr_ref.shape[0]
    xr = xr_ref[...]
    sqr = sqr_ref[...]
    br = br_ref[...]
    pcols = lax.broadcasted_iota(jnp.int32, (rb, 2 * K), 1).astype(jnp.float32)
    p_out = bc_ref[:, : 2 * K] != br
    pv = jnp.full((rb, 2 * K), jnp.inf, jnp.float32)
    pc = jnp.where(p_out, pcols, _FN)
    tv0 = jnp.full((rb, K), jnp.inf, jnp.float32)
    ti0 = jnp.full((rb, K), _FN, jnp.float32)
    tv, ti = _topk_merge(tv0, ti0, pv, pc)

    def body(c, carry):
        tv, ti = carry
        col0 = c * _CS
        xc = xt_ref[:, pl.ds(col0, _CS)]
        dot = jnp.dot(xr, xc, preferred_element_type=jnp.float32)
        dval = sqr + sqc_ref[:, pl.ds(col0, _CS)] - 2.0 * dot
        dval = jnp.where(br != bc_ref[:, pl.ds(col0, _CS)], jnp.inf, dval)
        dcol = (lax.broadcasted_iota(jnp.int32, (rb, _CS), 1).astype(jnp.float32)
                + col0.astype(jnp.float32))
        return _topk_merge(tv, ti, dval, dcol)

    tv, ti = lax.fori_loop(cs_ref[i], ce_ref[i], body, (tv, ti))
    idx_ref[...] = ti.astype(jnp.int32)


def _knn(x_bf, xt_bf, cs, ce, sq_row, sq_col, b_row, b_col):
    n = x_bf.shape[0]
    rb = 256
    return pl.pallas_call(
        _knn_body,
        grid=(n // rb,),
        in_specs=[
            pl.BlockSpec(memory_space=pltpu.SMEM),
            pl.BlockSpec(memory_space=pltpu.SMEM),
            pl.BlockSpec((rb, DP), lambda i: (i, 0)),
            pl.BlockSpec((DP, n), lambda i: (0, 0)),
            pl.BlockSpec((rb, 1), lambda i: (i, 0)),
            pl.BlockSpec((1, n), lambda i: (0, 0)),
            pl.BlockSpec((rb, 1), lambda i: (i, 0)),
            pl.BlockSpec((1, n), lambda i: (0, 0)),
        ],
        out_specs=pl.BlockSpec((rb, K), lambda i: (i, 0)),
        out_shape=jax.ShapeDtypeStruct((n, K), jnp.int32),
    )(cs, ce, x_bf, xt_bf, sq_row, sq_col, b_row, b_col)


def _sc_gather(table, idx_flat):
    n, h = table.shape
    m = idx_flat.shape[0]
    nw = 32
    per_w = m // nw
    rows_per_chunk = min(per_w, max(8, (128 * 1024) // (h * 4)))
    n_chunks = per_w // rows_per_chunk
    mesh = plsc.VectorSubcoreMesh(core_axis_name="c", subcore_axis_name="s")

    @functools.partial(
        pl.kernel,
        mesh=mesh,
        out_type=jax.ShapeDtypeStruct((m, h), jnp.float32),
        scratch_types=[
            pltpu.VMEM((rows_per_chunk,), jnp.int32),
            pltpu.VMEM((rows_per_chunk, h), jnp.float32),
            pltpu.SemaphoreType.DMA,
        ],
    )
    def k(tab_hbm, idx_hbm, out_hbm, idx_v, rows_v, sem):
        wid = lax.axis_index("s") * 2 + lax.axis_index("c")
        base = wid * per_w

        def body(c, carry):
            off = base + c * rows_per_chunk
            pltpu.sync_copy(idx_hbm.at[pl.ds(off, rows_per_chunk)], idx_v)
            pltpu.async_copy(tab_hbm.at[idx_v], rows_v, sem).wait()
            pltpu.sync_copy(rows_v, out_hbm.at[pl.ds(off, rows_per_chunk)])
            return carry

        lax.fori_loop(0, n_chunks, body, 0)

    return k(table, idx_flat)


def _mlp_out(x_ref, xg_ref, p_ref, w1b_ref, w2_ref, b2_ref):
    rb = x_ref.shape[0]
    h = p_ref.shape[1]
    o_dim = w2_ref.shape[1]
    delta = xg_ref[...].reshape(rb, K, DP) - x_ref[...][:, None, :]
    hid = _mm(delta.reshape(rb * K, DP), w1b_ref[...]).reshape(rb, K, h)
    hid = jnp.maximum(hid + p_ref[...][:, None, :], 0.0)
    hh = _mm(hid.reshape(rb * K, h), w2_ref[...])
    return jnp.max(hh.reshape(rb, K, o_dim), axis=1) + b2_ref[...]


def _mlp_body(x_ref, xg_ref, p_ref, w1b_ref, w2_ref, b2_ref, o_ref):
    o_ref[...] = _mlp_out(x_ref, xg_ref, p_ref, w1b_ref, w2_ref, b2_ref)


def _mlp_body_next(x_ref, xg_ref, p_ref, w1b_ref, w2_ref, b2_ref,
                   o_ref, xb_ref, xt_ref):
    o = _mlp_out(x_ref, xg_ref, p_ref, w1b_ref, w2_ref, b2_ref)
    o_dim = o.shape[1]
    if o_dim != DP:
        o = jnp.pad(o, ((0, 0), (0, DP - o_dim)))
    o_ref[...] = o
    xb = o.astype(_bf)
    xb_ref[...] = xb
    xt_ref[...] = xb.T


def _mlp(x, xg, p, w1b, w2, b2, emit_next):
    n, h = p.shape
    o_dim = w2.shape[1]
    rb = 128
    in_specs = [
        pl.BlockSpec((rb, DP), lambda i: (i, 0)),
        pl.BlockSpec((rb * K, DP), lambda i: (i, 0)),
        pl.BlockSpec((rb, h), lambda i: (i, 0)),
        pl.BlockSpec((DP, h), lambda i: (0, 0)),
        pl.BlockSpec((h, o_dim), lambda i: (0, 0)),
        pl.BlockSpec((1, o_dim), lambda i: (0, 0)),
    ]
    if not emit_next:
        return pl.pallas_call(
            _mlp_body,
            grid=(n // rb,),
            in_specs=in_specs,
            out_specs=pl.BlockSpec((rb, o_dim), lambda i: (i, 0)),
            out_shape=jax.ShapeDtypeStruct((n, o_dim), jnp.float32),
        )(x, xg, p, w1b, w2, b2)
    return pl.pallas_call(
        _mlp_body_next,
        grid=(n // rb,),
        in_specs=in_specs,
        out_specs=[
            pl.BlockSpec((rb, DP), lambda i: (i, 0)),
            pl.BlockSpec((rb, DP), lambda i: (i, 0)),
            pl.BlockSpec((DP, rb), lambda i: (0, i)),
        ],
        out_shape=[
            jax.ShapeDtypeStruct((n, DP), jnp.float32),
            jax.ShapeDtypeStruct((n, DP), _bf),
            jax.ShapeDtypeStruct((DP, n), _bf),
        ],
    )(x, xg, p, w1b, w2, b2)


def _head_body(x1_ref, x2_ref, x3_ref, x4_ref, bt_ref, w_ref, bv_ref,
               g_ref, be_ref, w2_ref, b2_ref, out_ref):
    xcat = jnp.concatenate(
        [x1_ref[:, :64], x2_ref[:, :64], x3_ref[...], x4_ref[...]], axis=1)
    bt = bt_ref[...]
    pooled = []
    for seg in range(B):
        pooled.append(
            jnp.max(jnp.where(bt == seg, xcat, -jnp.inf), axis=0,
                    keepdims=True))
    pooled = jnp.concatenate(pooled, axis=0)
    hh = _mm(pooled, w_ref[...]) + bv_ref[...]
    mu = jnp.mean(hh, axis=0, keepdims=True)
    var = jnp.mean((hh - mu) * (hh - mu), axis=0, keepdims=True)
    hn = g_ref[...] * (hh - mu) / jnp.sqrt(var + 1e-5) + be_ref[...]
    hn = jnp.maximum(hn, 0.0)
    logits = _mm(hn, w2_ref[...]) + b2_ref[...]
    mx = jnp.max(logits, axis=1, keepdims=True)
    sh = logits - mx
    out_ref[...] = sh - jnp.log(jnp.sum(jnp.exp(sh), axis=1, keepdims=True))


def _head(x1, x2, x3, x4, bt, w, bv, g, be, w2, b2):
    nc = w2.shape[1]
    emb = w.shape[1]
    return pl.pallas_call(
        _head_body,
        in_specs=[pl.BlockSpec(x.shape, lambda: (0, 0))
                  for x in (x1, x2, x3, x4)]
        + [
            pl.BlockSpec((N, 1), lambda: (0, 0)),
            pl.BlockSpec((512, emb), lambda: (0, 0)),
            pl.BlockSpec((1, emb), lambda: (0, 0)),
            pl.BlockSpec((1, emb), lambda: (0, 0)),
            pl.BlockSpec((1, emb), lambda: (0, 0)),
            pl.BlockSpec((emb, nc), lambda: (0, 0)),
            pl.BlockSpec((1, nc), lambda: (0, 0)),
        ],
        out_specs=pl.BlockSpec((B, nc), lambda: (0, 0)),
        out_shape=jax.ShapeDtypeStruct((B, nc), jnp.float32),
    )(x1, x2, x3, x4, bt, w, bv, g, be, w2, b2)


def _edge_conv(xp, xb, xt, d, seg, p_conv, emit_next):
    b_row, b_col, cs, ce = seg
    w1 = p_conv["W1"]
    w1a, w1b = w1[:d], w1[d:]
    if d != DP:
        w1a = jnp.pad(w1a, ((0, DP - d), (0, 0)))
        w1b = jnp.pad(w1b, ((0, DP - d), (0, 0)))
    p, sq = _p_and_sq(xp, w1a, p_conv["b1"][None, :])
    idx = _knn(xb, xt, cs, ce, sq, sq.reshape(1, N), b_row, b_col)
    xg = _sc_gather(xp, idx.reshape(N * K))
    return _mlp(xp, xg, p, w1b, p_conv["W2"], p_conv["b2"][None, :],
                emit_next)


def kernel(pos, params, batch):
    bt = batch.astype(jnp.int32)
    b_row = bt.reshape(N, 1)
    b_col = bt.reshape(1, N)
    segs = jnp.arange(B, dtype=jnp.int32)
    starts = jnp.searchsorted(bt, segs, side="left")
    ends = jnp.searchsorted(bt, segs, side="right")
    cs = (starts[bt[::256]] // _CS).astype(jnp.int32)
    ce = ((ends[bt[255::256]] + _CS - 1) // _CS).astype(jnp.int32)
    seg = (b_row, b_col, cs, ce)
    xp = jnp.pad(pos, ((0, 0), (0, DP - pos.shape[1])))
    xb = xp.astype(_bf)
    x1, x1b, x1t = _edge_conv(xp, xb, xb.T, 3, seg, params["conv1"], True)
    x2, x2b, x2t = _edge_conv(x1, x1b, x1t, 64, seg, params["conv2"], True)
    x3, x3b, x3t = _edge_conv(x2, x2b, x2t, 64, seg, params["conv3"], True)
    x4 = _edge_conv(x3, x3b, x3t, 128, seg, params["conv4"], False)
    return _head(
        x1, x2, x3, x4, b_row,
        params["lin1"]["W"], params["lin1"]["b"][None, :],
        params["bn1"]["gamma"][None, :], params["bn1"]["beta"][None, :],
        params["lin2"]["W"], params["lin2"]["b"][None, :],
    )


# speedup vs baseline: 17.5856x; 1.0275x over previous
import functools

import jax
import jax.numpy as jnp
from jax import lax
from jax.experimental import pallas as pl
from jax.experimental.pallas import tpu as pltpu
from jax.experimental.pallas import tpu_sc as plsc

N = 8192
B = 8
K = 20
DP = 128

_bf = jnp.bfloat16


def _mm(a, b):
    return jnp.dot(a.astype(_bf), b.astype(_bf),
                   preferred_element_type=jnp.float32)


def _p_body(x_ref, w1a_ref, b1_ref, p_ref, sq_ref):
    x = x_ref[...]
    p_ref[...] = _mm(x, w1a_ref[...]) + b1_ref[...]
    sq_ref[...] = jnp.sum(x * x, axis=1, keepdims=True)


def _p_and_sq(x, w1a, b1):
    n = x.shape[0]
    h = w1a.shape[1]
    rb = 1024
    return pl.pallas_call(
        _p_body,
        grid=(n // rb,),
        in_specs=[
            pl.BlockSpec((rb, DP), lambda i: (i, 0)),
            pl.BlockSpec((DP, h), lambda i: (0, 0)),
            pl.BlockSpec((1, h), lambda i: (0, 0)),
        ],
        out_specs=[
            pl.BlockSpec((rb, h), lambda i: (i, 0)),
            pl.BlockSpec((rb, 1), lambda i: (i, 0)),
        ],
        out_shape=[
            jax.ShapeDtypeStruct((n, h), jnp.float32),
            jax.ShapeDtypeStruct((n, 1), jnp.float32),
        ],
    )(x, w1a, b1)


_CS = 512


_FN = float(N)


def _topk_merge(tv, ti, cv, cc):
    picks_v, picks_i = [], []
    for _ in range(K):
        m = jnp.minimum(jnp.min(tv, axis=1, keepdims=True),
                        jnp.min(cv, axis=1, keepdims=True))
        jt = jnp.min(jnp.where(tv == m, ti, _FN), axis=1, keepdims=True)
        jc = jnp.min(jnp.where(cv == m, cc, _FN), axis=1, keepdims=True)
        j = jnp.minimum(jt, jc)
        picks_v.append(m)
        picks_i.append(j)
        tv = jnp.where(ti == j, jnp.inf, tv)
        cv = jnp.where(cc == j, jnp.inf, cv)
    return (jnp.concatenate(picks_v, axis=1),
            jnp.concatenate(picks_i, axis=1))


def _knn_body(cs_ref, ce_ref, xr_ref, xt_ref, sqr_ref, sqc_ref, br_ref,
              bc_ref, idx_ref):
    i = pl.program_id(0)
    rb = xr_ref.shape[0]
    xr = xr_ref[...]
    sqr = sqr_ref[...]
    br = br_ref[...]
    pcols = lax.broadcasted_iota(jnp.int32, (rb, 2 * K), 1).astype(jnp.float32)
    p_out = bc_ref[:, : 2 * K] != br
    pv = jnp.full((rb, 2 * K), jnp.inf, jnp.float32)
    pc = jnp.where(p_out, pcols, _FN)
    tv0 = jnp.full((rb, K), jnp.inf, jnp.float32)
    ti0 = jnp.full((rb, K), _FN, jnp.float32)
    tv, ti = _topk_merge(tv0, ti0, pv, pc)

    def body(c, carry):
        tv, ti = carry
        col0 = c * _CS
        xc = xt_ref[:, pl.ds(col0, _CS)]
        dot = jnp.dot(xr, xc, preferred_element_type=jnp.float32)
        dval = sqr + sqc_ref[:, pl.ds(col0, _CS)] - 2.0 * dot
        dval = jnp.where(br != bc_ref[:, pl.ds(col0, _CS)], jnp.inf, dval)
        dcol = (lax.broadcasted_iota(jnp.int32, (rb, _CS), 1).astype(jnp.float32)
                + col0.astype(jnp.float32))
        return _topk_merge(tv, ti, dval, dcol)

    tv, ti = lax.fori_loop(cs_ref[i], ce_ref[i], body, (tv, ti))
    idx_ref[...] = ti.astype(jnp.int32)


def _knn(x_bf, xt_bf, cs, ce, sq_row, sq_col, b_row, b_col):
    n = x_bf.shape[0]
    rb = 256
    return pl.pallas_call(
        _knn_body,
        grid=(n // rb,),
        in_specs=[
            pl.BlockSpec(memory_space=pltpu.SMEM),
            pl.BlockSpec(memory_space=pltpu.SMEM),
            pl.BlockSpec((rb, DP), lambda i: (i, 0)),
            pl.BlockSpec((DP, n), lambda i: (0, 0)),
            pl.BlockSpec((rb, 1), lambda i: (i, 0)),
            pl.BlockSpec((1, n), lambda i: (0, 0)),
            pl.BlockSpec((rb, 1), lambda i: (i, 0)),
            pl.BlockSpec((1, n), lambda i: (0, 0)),
        ],
        out_specs=pl.BlockSpec((rb, K), lambda i: (i, 0)),
        out_shape=jax.ShapeDtypeStruct((n, K), jnp.int32),
    )(cs, ce, x_bf, xt_bf, sq_row, sq_col, b_row, b_col)


def _sc_gather(table, idx_flat):
    n, h = table.shape
    m = idx_flat.shape[0]
    nw = 32
    per_w = m // nw
    rows_per_chunk = min(per_w, max(8, (128 * 1024) // (h * 4)))
    n_chunks = per_w // rows_per_chunk
    mesh = plsc.VectorSubcoreMesh(core_axis_name="c", subcore_axis_name="s")

    @functools.partial(
        pl.kernel,
        mesh=mesh,
        out_type=jax.ShapeDtypeStruct((m, h), jnp.float32),
        scratch_types=[
            pltpu.VMEM((rows_per_chunk,), jnp.int32),
            pltpu.VMEM((rows_per_chunk, h), jnp.float32),
            pltpu.SemaphoreType.DMA,
        ],
    )
    def k(tab_hbm, idx_hbm, out_hbm, idx_v, rows_v, sem):
        wid = lax.axis_index("s") * 2 + lax.axis_index("c")
        base = wid * per_w

        def body(c, carry):
            off = base + c * rows_per_chunk
            pltpu.sync_copy(idx_hbm.at[pl.ds(off, rows_per_chunk)], idx_v)
            pltpu.async_copy(tab_hbm.at[idx_v], rows_v, sem).wait()
            pltpu.sync_copy(rows_v, out_hbm.at[pl.ds(off, rows_per_chunk)])
            return carry

        lax.fori_loop(0, n_chunks, body, 0)

    return k(table, idx_flat)


def _mlp_out(x_ref, xg_ref, p_ref, w1b_ref, w2_ref, b2_ref):
    rb = x_ref.shape[0]
    h = p_ref.shape[1]
    o_dim = w2_ref.shape[1]
    delta = xg_ref[...].reshape(rb, K, DP) - x_ref[...][:, None, :]
    hid = _mm(delta.reshape(rb * K, DP), w1b_ref[...]).reshape(rb, K, h)
    hid = jnp.maximum(hid + p_ref[...][:, None, :], 0.0)
    hh = _mm(hid.reshape(rb * K, h), w2_ref[...])
    return jnp.max(hh.reshape(rb, K, o_dim), axis=1) + b2_ref[...]


def _mlp_body(x_ref, xg_ref, p_ref, w1b_ref, w2_ref, b2_ref, o_ref):
    o_ref[...] = _mlp_out(x_ref, xg_ref, p_ref, w1b_ref, w2_ref, b2_ref)


def _mlp_body_next(x_ref, xg_ref, p_ref, w1b_ref, w2_ref, b2_ref,
                   o_ref, xb_ref, xt_ref):
    o = _mlp_out(x_ref, xg_ref, p_ref, w1b_ref, w2_ref, b2_ref)
    o_dim = o.shape[1]
    if o_dim != DP:
        o = jnp.pad(o, ((0, 0), (0, DP - o_dim)))
    o_ref[...] = o
    xb = o.astype(_bf)
    xb_ref[...] = xb
    xt_ref[...] = xb.T


def _mlp(x, xg, p, w1b, w2, b2, emit_next):
    n, h = p.shape
    o_dim = w2.shape[1]
    rb = 128
    in_specs = [
        pl.BlockSpec((rb, DP), lambda i: (i, 0)),
        pl.BlockSpec((rb * K, DP), lambda i: (i, 0)),
        pl.BlockSpec((rb, h), lambda i: (i, 0)),
        pl.BlockSpec((DP, h), lambda i: (0, 0)),
        pl.BlockSpec((h, o_dim), lambda i: (0, 0)),
        pl.BlockSpec((1, o_dim), lambda i: (0, 0)),
    ]
    if not emit_next:
        return pl.pallas_call(
            _mlp_body,
            grid=(n // rb,),
            in_specs=in_specs,
            out_specs=pl.BlockSpec((rb, o_dim), lambda i: (i, 0)),
            out_shape=jax.ShapeDtypeStruct((n, o_dim), jnp.float32),
        )(x, xg, p, w1b, w2, b2)
    return pl.pallas_call(
        _mlp_body_next,
        grid=(n // rb,),
        in_specs=in_specs,
        out_specs=[
            pl.BlockSpec((rb, DP), lambda i: (i, 0)),
            pl.BlockSpec((rb, DP), lambda i: (i, 0)),
            pl.BlockSpec((DP, rb), lambda i: (0, i)),
        ],
        out_shape=[
            jax.ShapeDtypeStruct((n, DP), jnp.float32),
            jax.ShapeDtypeStruct((n, DP), _bf),
            jax.ShapeDtypeStruct((DP, n), _bf),
        ],
    )(x, xg, p, w1b, w2, b2)


def _head_body(x1_ref, x2_ref, x3_ref, x4_ref, bt_ref, w_ref, bv_ref,
               g_ref, be_ref, w2_ref, b2_ref, out_ref):
    xcat = jnp.concatenate(
        [x1_ref[:, :64], x2_ref[:, :64], x3_ref[...], x4_ref[...]], axis=1)
    bt = bt_ref[...]
    pooled = []
    for seg in range(B):
        pooled.append(
            jnp.max(jnp.where(bt == seg, xcat, -jnp.inf), axis=0,
                    keepdims=True))
    pooled = jnp.concatenate(pooled, axis=0)
    hh = _mm(pooled, w_ref[...]) + bv_ref[...]
    mu = jnp.mean(hh, axis=0, keepdims=True)
    var = jnp.mean((hh - mu) * (hh - mu), axis=0, keepdims=True)
    hn = g_ref[...] * (hh - mu) / jnp.sqrt(var + 1e-5) + be_ref[...]
    hn = jnp.maximum(hn, 0.0)
    logits = _mm(hn, w2_ref[...]) + b2_ref[...]
    mx = jnp.max(logits, axis=1, keepdims=True)
    sh = logits - mx
    out_ref[...] = sh - jnp.log(jnp.sum(jnp.exp(sh), axis=1, keepdims=True))


def _head(x1, x2, x3, x4, bt, w, bv, g, be, w2, b2):
    nc = w2.shape[1]
    emb = w.shape[1]
    return pl.pallas_call(
        _head_body,
        in_specs=[pl.BlockSpec(x.shape, lambda: (0, 0))
                  for x in (x1, x2, x3, x4)]
        + [
            pl.BlockSpec((N, 1), lambda: (0, 0)),
            pl.BlockSpec((512, emb), lambda: (0, 0)),
            pl.BlockSpec((1, emb), lambda: (0, 0)),
            pl.BlockSpec((1, emb), lambda: (0, 0)),
            pl.BlockSpec((1, emb), lambda: (0, 0)),
            pl.BlockSpec((emb, nc), lambda: (0, 0)),
            pl.BlockSpec((1, nc), lambda: (0, 0)),
        ],
        out_specs=pl.BlockSpec((B, nc), lambda: (0, 0)),
        out_shape=jax.ShapeDtypeStruct((B, nc), jnp.float32),
    )(x1, x2, x3, x4, bt, w, bv, g, be, w2, b2)


def _edge_conv(xp, xb, xt, d, seg, p_conv, emit_next):
    b_row, b_col, cs, ce = seg
    w1 = p_conv["W1"]
    w1a, w1b = w1[:d], w1[d:]
    if d != DP:
        w1a = jnp.pad(w1a, ((0, DP - d), (0, 0)))
        w1b = jnp.pad(w1b, ((0, DP - d), (0, 0)))
    p, sq = _p_and_sq(xp, w1a, p_conv["b1"][None, :])
    idx = _knn(xb, xt, cs, ce, sq, sq.reshape(1, N), b_row, b_col)
    idx_flat = idx.reshape(N * K)
    hn = N // 2
    w2 = p_conv["W2"]
    b2 = p_conv["b2"][None, :]
    xg_a = _sc_gather(xp, idx_flat[: hn * K])
    xg_b = _sc_gather(xp, idx_flat[hn * K:])
    out_a = _mlp(xp[:hn], xg_a, p[:hn], w1b, w2, b2, emit_next)
    out_b = _mlp(xp[hn:], xg_b, p[hn:], w1b, w2, b2, emit_next)
    if not emit_next:
        return jnp.concatenate([out_a, out_b], axis=0)
    return (jnp.concatenate([out_a[0], out_b[0]], axis=0),
            jnp.concatenate([out_a[1], out_b[1]], axis=0),
            jnp.concatenate([out_a[2], out_b[2]], axis=1))


def kernel(pos, params, batch):
    bt = batch.astype(jnp.int32)
    b_row = bt.reshape(N, 1)
    b_col = bt.reshape(1, N)
    segs = jnp.arange(B, dtype=jnp.int32)
    starts = jnp.searchsorted(bt, segs, side="left")
    ends = jnp.searchsorted(bt, segs, side="right")
    cs = (starts[bt[::256]] // _CS).astype(jnp.int32)
    ce = ((ends[bt[255::256]] + _CS - 1) // _CS).astype(jnp.int32)
    seg = (b_row, b_col, cs, ce)
    xp = jnp.pad(pos, ((0, 0), (0, DP - pos.shape[1])))
    xb = xp.astype(_bf)
    x1, x1b, x1t = _edge_conv(xp, xb, xb.T, 3, seg, params["conv1"], True)
    x2, x2b, x2t = _edge_conv(x1, x1b, x1t, 64, seg, params["conv2"], True)
    x3, x3b, x3t = _edge_conv(x2, x2b, x2t, 64, seg, params["conv3"], True)
    x4 = _edge_conv(x3, x3b, x3t, 128, seg, params["conv4"], False)
    return _head(
        x1, x2, x3, x4, b_row,
        params["lin1"]["W"], params["lin1"]["b"][None, :],
        params["bn1"]["gamma"][None, :], params["bn1"]["beta"][None, :],
        params["lin2"]["W"], params["lin2"]["b"][None, :],
    )
